# baseline (speedup 1.0000x reference)
import jax
import jax.numpy as jnp
from jax.experimental import pallas as pl
from jax.experimental.pallas import tpu as pltpu

TOKENS = 32768
HIDDEN = 4096
EXPERTS = 64
BLOCK_T = 512
NBLOCKS = TOKENS // BLOCK_T


def _softmax_rows(logits):
    m = jnp.max(logits, axis=-1, keepdims=True)
    e = jnp.exp(logits - m)
    return e / jnp.sum(e, axis=-1, keepdims=True)


def _gating_body(xa_ref, xb_ref, w_ref, b_ref, o_ref):
    wt = w_ref[...]
    bias = b_ref[...]
    la = jax.lax.dot_general(
        xa_ref[0], wt, dimension_numbers=(((1,), (1,)), ((), ())),
        preferred_element_type=jnp.float32,
    ) + bias
    lb = jax.lax.dot_general(
        xb_ref[0], wt, dimension_numbers=(((1,), (1,)), ((), ())),
        preferred_element_type=jnp.float32,
    ) + bias
    o_ref[0] = _softmax_rows(la)
    o_ref[1] = _softmax_rows(lb)


def kernel(x, W, b):
    b2 = b.reshape(1, EXPERTS)
    x4 = x.reshape(NBLOCKS, BLOCK_T, HIDDEN)
    grid = (NBLOCKS // 2,)
    out = pl.pallas_call(
        _gating_body,
        grid=grid,
        in_specs=[
            pl.BlockSpec((1, BLOCK_T, HIDDEN), lambda i: (2 * i, 0, 0)),
            pl.BlockSpec((1, BLOCK_T, HIDDEN), lambda i: (2 * i + 1, 0, 0)),
            pl.BlockSpec((EXPERTS, HIDDEN), lambda i: (0, 0)),
            pl.BlockSpec((1, EXPERTS), lambda i: (0, 0)),
        ],
        out_specs=pl.BlockSpec((2, BLOCK_T, EXPERTS), lambda i: (i, 0, 0)),
        out_shape=jax.ShapeDtypeStruct((NBLOCKS, BLOCK_T, EXPERTS), jnp.float32),
        compiler_params=pltpu.CompilerParams(
            dimension_semantics=("arbitrary",),
        ),
    )(x4, x4, W, b2)
    return out.reshape(TOKENS, EXPERTS)


# speedup vs baseline: 1.0578x; 1.0578x over previous
import jax
import jax.numpy as jnp
from jax.experimental import pallas as pl
from jax.experimental.pallas import tpu as pltpu

TOKENS = 32768
HIDDEN = 4096
EXPERTS = 64
BLOCK_T = 512
XBUFS = 4


def _inner(x_ref, o_ref, w_ref, b_ref):
    logits = jax.lax.dot_general(
        x_ref[...], w_ref[...],
        dimension_numbers=(((1,), (1,)), ((), ())),
        preferred_element_type=jnp.float32,
    )
    logits = logits + b_ref[...]
    m = jnp.max(logits, axis=-1, keepdims=True)
    e = jnp.exp(logits - m)
    o_ref[...] = e / jnp.sum(e, axis=-1, keepdims=True)


def _outer(x_hbm, w_ref, b_ref, o_hbm):
    w = w_ref
    bias = b_ref

    def body(x_ref, o_ref):
        _inner(x_ref, o_ref, w, bias)

    pipeline = pltpu.emit_pipeline(
        body,
        grid=(TOKENS // BLOCK_T,),
        in_specs=[
            pl.BlockSpec((BLOCK_T, HIDDEN), lambda i: (i, 0),
                         pipeline_mode=pl.Buffered(buffer_count=XBUFS)),
        ],
        out_specs=[
            pl.BlockSpec((BLOCK_T, EXPERTS), lambda i: (i, 0)),
        ],
    )
    pipeline(x_hbm, o_hbm)


def kernel(x, W, b):
    b2 = b.reshape(1, EXPERTS)
    return pl.pallas_call(
        _outer,
        in_specs=[
            pl.BlockSpec(memory_space=pltpu.MemorySpace.HBM),
            pl.BlockSpec((EXPERTS, HIDDEN), lambda: (0, 0)),
            pl.BlockSpec((1, EXPERTS), lambda: (0, 0)),
        ],
        out_specs=pl.BlockSpec(memory_space=pltpu.MemorySpace.HBM),
        out_shape=jax.ShapeDtypeStruct((TOKENS, EXPERTS), jnp.float32),
    )(x, W, b2)
